# baseline (speedup 1.0000x reference)
import jax
import jax.numpy as jnp
from jax import lax
from jax.experimental import pallas as pl
from jax.experimental.pallas import tpu as pltpu

N = 320000
D = 128
S = 10000
BR = 4000
NBLK = N // BR
W = 144
SPAD = S + 2 * W


def _tc_body(idx_ref, x_ref, out_ref, acc_ref, cnt_ref):
    i = pl.program_id(0)

    @pl.when(i == 0)
    def _():
        acc_ref[...] = jnp.zeros_like(acc_ref)
        cnt_ref[...] = jnp.zeros_like(cnt_ref)

    idx = idx_ref[0, 0, :]
    x = x_ref[...]
    x2 = (x * x).astype(jnp.bfloat16)
    ones = jnp.ones((BR, 8), jnp.bfloat16)

    mn = jnp.min(idx)
    mx = jnp.max(idx)
    base0 = (mn // 8) * 8
    npass = (mx + 1 - base0 + (W - 1)) // W

    def pass_body(k, c):
        base = base0 + k * W
        lseg = idx - base
        oh = (lseg[:, None] ==
              lax.broadcasted_iota(jnp.int32, (BR, W), 1)).astype(jnp.bfloat16)
        dn = (((0,), (0,)), ((), ()))
        sums = lax.dot_general(oh, x2, dn, preferred_element_type=jnp.float32)
        cnts = lax.dot_general(oh, ones, dn, preferred_element_type=jnp.float32)
        acc_ref[pl.ds(base, W), :] += sums
        cnt_ref[pl.ds(base, W), :] += cnts
        return c

    lax.fori_loop(0, npass, pass_body, 0)

    @pl.when(i == NBLK - 1)
    def _():
        sums = acc_ref[pl.ds(0, S), :]
        cnt = cnt_ref[pl.ds(0, S), pl.ds(0, 8)][:, 0:1]
        out_ref[...] = jnp.sqrt(sums / jnp.maximum(cnt, 1.0))


@jax.jit
def _run(x, index):
    idx3 = index.reshape(NBLK, 1, BR)
    return pl.pallas_call(
        _tc_body,
        grid=(NBLK,),
        in_specs=[
            pl.BlockSpec((1, 1, BR), lambda i: (i, 0, 0)),
            pl.BlockSpec((BR, D), lambda i: (i, 0)),
        ],
        out_specs=pl.BlockSpec((S, D), lambda i: (0, 0)),
        out_shape=jax.ShapeDtypeStruct((S, D), jnp.float32),
        scratch_shapes=[pltpu.VMEM((SPAD, D), jnp.float32),
                        pltpu.VMEM((SPAD, 8), jnp.float32)],
    )(idx3, x)


def kernel(x, index, dim_size):
    return _run(x, index)


# speedup vs baseline: 1.0696x; 1.0696x over previous
import jax
import jax.numpy as jnp
from jax import lax
from jax.experimental import pallas as pl
from jax.experimental.pallas import tpu as pltpu

N = 320000
D = 128
S = 10000
BR = 3200
NBLK = N // BR
W = 120
SPAD = S + 2 * W


def _tc_body(idx_ref, x_ref, out_ref, acc_ref, cnt_ref):
    i = pl.program_id(0)

    @pl.when(i == 0)
    def _():
        acc_ref[...] = jnp.zeros_like(acc_ref)
        cnt_ref[...] = jnp.zeros_like(cnt_ref)

    idx = idx_ref[0, 0, :]
    x = x_ref[...]
    x2 = (x * x).astype(jnp.bfloat16)
    ones = jnp.ones((BR, 8), jnp.bfloat16)

    mn = jnp.min(idx)
    mx = jnp.max(idx)
    base0 = (mn // 8) * 8
    npass = (mx + 1 - base0 + (W - 1)) // W

    def pass_body(k, c):
        base = base0 + k * W
        lseg = idx - base
        oh = (lseg[:, None] ==
              lax.broadcasted_iota(jnp.int32, (BR, W), 1)).astype(jnp.bfloat16)
        dn = (((0,), (0,)), ((), ()))
        sums = lax.dot_general(oh, x2, dn, preferred_element_type=jnp.float32)
        cnts = lax.dot_general(oh, ones, dn, preferred_element_type=jnp.float32)
        acc_ref[pl.ds(base, W), :] += sums
        cnt_ref[pl.ds(base, W), :] += cnts
        return c

    lax.fori_loop(0, npass, pass_body, 0)

    @pl.when(i == NBLK - 1)
    def _():
        sums = acc_ref[pl.ds(0, S), :]
        cnt = cnt_ref[pl.ds(0, S), pl.ds(0, 8)][:, 0:1]
        out_ref[...] = jnp.sqrt(sums / jnp.maximum(cnt, 1.0))


@jax.jit
def _run(x, index):
    idx3 = index.reshape(NBLK, 1, BR)
    return pl.pallas_call(
        _tc_body,
        grid=(NBLK,),
        in_specs=[
            pl.BlockSpec((1, 1, BR), lambda i: (i, 0, 0)),
            pl.BlockSpec((BR, D), lambda i: (i, 0)),
        ],
        out_specs=pl.BlockSpec((S, D), lambda i: (0, 0)),
        out_shape=jax.ShapeDtypeStruct((S, D), jnp.float32),
        scratch_shapes=[pltpu.VMEM((SPAD, D), jnp.float32),
                        pltpu.VMEM((SPAD, 8), jnp.float32)],
    )(idx3, x)


def kernel(x, index, dim_size):
    return _run(x, index)
